# Initial kernel scaffold; baseline (speedup 1.0000x reference)
import jax
import jax.numpy as jnp
from jax.experimental import pallas as pl


def kernel(features, neighbor_indices):
    raise NotImplementedError("write your pallas kernel here")


# speedup vs baseline: 11281.9753x; 11281.9753x over previous
import functools

import jax
import jax.numpy as jnp
from jax import lax
from jax.experimental import pallas as pl
from jax.experimental.pallas import tpu as pltpu
from jax.experimental.pallas import tpu_sc as plsc

B = 160000
D = 256
V = 10000
C = 128
NC = 2
NS = 16
NW = NC * NS
NCHUNKS = B // C


def _gather_body(table_hbm, idx_hbm, out_hbm, idx_v, rows_v, gsem):
    wid = lax.axis_index("s") * NC + lax.axis_index("c")
    n_base = NCHUNKS // NW
    rem = NCHUNKS % NW
    n_mine = n_base + jnp.where(wid < rem, 1, 0)

    def step(g, carry):
        off = (g * NW + wid) * C
        pltpu.sync_copy(idx_hbm.at[pl.ds(off, C)], idx_v)
        pltpu.async_copy(table_hbm.at[idx_v], rows_v, gsem).wait()
        pltpu.sync_copy(rows_v, out_hbm.at[pl.ds(off, C)])
        return carry

    lax.fori_loop(0, n_mine, step, 0)


_sc_gather = functools.partial(
    pl.kernel,
    out_type=jax.ShapeDtypeStruct((B, D), jnp.float32),
    mesh=plsc.VectorSubcoreMesh(core_axis_name="c", subcore_axis_name="s"),
    scratch_types=[
        pltpu.VMEM((C,), jnp.int32),
        pltpu.VMEM((C, D), jnp.float32),
        pltpu.SemaphoreType.DMA,
    ],
)(_gather_body)


def kernel(features, neighbor_indices):
    table = features[0]
    idx = neighbor_indices.reshape(B)
    out = _sc_gather(table, idx)
    return out[None]


# speedup vs baseline: 14866.0491x; 1.3177x over previous
import functools

import jax
import jax.numpy as jnp
from jax import lax
from jax.experimental import pallas as pl
from jax.experimental.pallas import tpu as pltpu
from jax.experimental.pallas import tpu_sc as plsc

B = 160000
D = 256
NC = 2
NS = 16
NW = NC * NS
BPW = B // NW
C = 200
NCH = BPW // C


def _gather_body(table_hbm, idx_hbm, out_hbm, idx_v, rows0, rows1,
                 gsem0, gsem1, wsem0, wsem1):
    wid = lax.axis_index("s") * NC + lax.axis_index("c")
    base = wid * BPW
    rows = (rows0, rows1)
    gsem = (gsem0, gsem1)
    wsem = (wsem0, wsem1)

    pltpu.sync_copy(idx_hbm.at[pl.ds(base, BPW)], idx_v)

    def gather_desc(slot, g):
        return pltpu.make_async_copy(
            table_hbm.at[idx_v.at[pl.ds(g * C, C)]], rows[slot], gsem[slot])

    def write_desc(slot, g):
        return pltpu.make_async_copy(
            rows[slot], out_hbm.at[pl.ds(base + g * C, C)], wsem[slot])

    gather_desc(0, 0).start()

    def outer(k, carry):
        for b in range(2):
            g = 2 * k + b

            @pl.when((g > 0) & (g < NCH))
            def _():
                write_desc(1 - b, g - 1).wait()

            @pl.when(g + 1 < NCH)
            def _():
                gather_desc(1 - b, g + 1).start()

            @pl.when(g < NCH)
            def _():
                gather_desc(b, g).wait()
                write_desc(b, g).start()
        return carry

    lax.fori_loop(0, (NCH + 1) // 2, outer, 0)
    write_desc(0, NCH - 1).wait()


_sc_gather = functools.partial(
    pl.kernel,
    out_type=jax.ShapeDtypeStruct((B, D), jnp.float32),
    mesh=plsc.VectorSubcoreMesh(core_axis_name="c", subcore_axis_name="s"),
    scratch_types=[
        pltpu.VMEM((BPW,), jnp.int32),
        pltpu.VMEM((C, D), jnp.float32),
        pltpu.VMEM((C, D), jnp.float32),
        pltpu.SemaphoreType.DMA,
        pltpu.SemaphoreType.DMA,
        pltpu.SemaphoreType.DMA,
        pltpu.SemaphoreType.DMA,
    ],
)(_gather_body)


def kernel(features, neighbor_indices):
    table = features[0]
    idx = neighbor_indices.reshape(B)
    out = _sc_gather(table, idx)
    return out[None]
